# Initial kernel scaffold; baseline (speedup 1.0000x reference)
import jax
import jax.numpy as jnp
from jax.experimental import pallas as pl


def kernel(x, params, edge_index, batch):
    raise NotImplementedError("write your pallas kernel here")


# speedup vs baseline: 12.3890x; 12.3890x over previous
import numpy as np
import jax
import jax.numpy as jnp
from jax import lax
from jax.experimental import pallas as pl
from jax.experimental.pallas import tpu as pltpu
from jax.experimental.pallas import tpu_sc as plsc

N = 10000
E = 160000
G = 256
HC3 = 512
HC4 = 256
NW = 32
NS = 80
NSTRIP = 128
NPAD = NS * NSTRIP
KC = 128
KN = 256
EPAD = E + 512
BM = 256
INV_BN = float(1.0 / np.sqrt(1.0 + 1e-5))

_i32 = jnp.int32
_f32 = jnp.float32


def _elu(v):
  return jnp.where(v > 0, v, jnp.exp(v) - 1.0)


def _tc_first(xp, w0, w1p, asf, adf, sel, g0, b0):
  def body(x_ref, w0_ref, w1_ref, asf_ref, adf_ref, sel_ref, g_ref, b_ref,
           h0_ref, h1_ref, h2_ref, h3_ref, as_ref, ad_ref):
    h0 = jnp.dot(x_ref[...], w0_ref[...], preferred_element_type=_f32)
    h0 = _elu(h0 * INV_BN * g_ref[...] + b_ref[...])
    h1 = jnp.dot(h0, w1_ref[...], preferred_element_type=_f32)
    h0_ref[...] = h1[:, 0:128]
    h1_ref[...] = h1[:, 128:256]
    h2_ref[...] = h1[:, 256:384]
    h3_ref[...] = h1[:, 384:512]
    as_ref[...] = jnp.dot(h1 * asf_ref[...], sel_ref[...],
                          preferred_element_type=_f32)
    ad_ref[...] = jnp.dot(h1 * adf_ref[...], sel_ref[...],
                          preferred_element_type=_f32)

  grid = (NPAD // BM,)
  bs_full = lambda shp: pl.BlockSpec(shp, lambda i: (0, 0))
  outs = [jax.ShapeDtypeStruct((NPAD, 128), _f32)] * 6
  return pl.pallas_call(
      body,
      grid=grid,
      in_specs=[
          pl.BlockSpec((BM, 128), lambda i: (i, 0)),
          bs_full((128, 128)), bs_full((128, HC3)),
          bs_full((1, HC3)), bs_full((1, HC3)), bs_full((HC3, 128)),
          bs_full((1, 128)), bs_full((1, 128)),
      ],
      out_specs=[pl.BlockSpec((BM, 128), lambda i: (i, 0))] * 6,
      out_shape=outs,
  )(xp, w0, w1p, asf, adf, sel, g0, b0)


def _tc_mid(aggs, resid, bias, g, b, w, asf, adf, sel, hcout, nco):
  has_resid = resid is not None

  def body(*refs):
    idx = 0
    agg_refs = refs[idx:idx + 4]; idx += 4
    if has_resid:
      r_ref = refs[idx]; idx += 1
    bias_ref, g_ref, b_ref, w_ref, asf_ref, adf_ref, sel_ref = refs[idx:idx + 7]
    idx += 7
    x_ref = refs[idx]; idx += 1
    h_refs = refs[idx:idx + nco]; idx += nco
    as_ref, ad_ref = refs[idx:idx + 2]
    agg = jnp.concatenate([r[...] for r in agg_refs], axis=1)
    xi = _elu((agg + bias_ref[...]) * INV_BN * g_ref[...] + b_ref[...])
    if has_resid:
      xi = xi + r_ref[...]
    x_ref[...] = xi
    h = jnp.dot(xi, w_ref[...], preferred_element_type=_f32)
    for c in range(nco):
      h_refs[c][...] = h[:, c * 128:(c + 1) * 128]
    as_ref[...] = jnp.dot(h * asf_ref[...], sel_ref[...],
                          preferred_element_type=_f32)
    ad_ref[...] = jnp.dot(h * adf_ref[...], sel_ref[...],
                          preferred_element_type=_f32)

  grid = (NPAD // BM,)
  bs_full = lambda shp: pl.BlockSpec(shp, lambda i: (0, 0))
  in_specs = [pl.BlockSpec((BM, 128), lambda i: (i, 0))] * 4
  operands = list(aggs)
  if has_resid:
    in_specs.append(pl.BlockSpec((BM, HC3), lambda i: (i, 0)))
    operands.append(resid)
  in_specs += [bs_full((1, HC3)), bs_full((1, HC3)), bs_full((1, HC3)),
               bs_full((HC3, hcout)), bs_full((1, hcout)), bs_full((1, hcout)),
               bs_full((hcout, 128))]
  operands += [bias, g, b, w, asf, adf, sel]
  out_shape = ([jax.ShapeDtypeStruct((NPAD, HC3), _f32)]
               + [jax.ShapeDtypeStruct((NPAD, 128), _f32)] * (nco + 2))
  out_specs = ([pl.BlockSpec((BM, HC3), lambda i: (i, 0))]
               + [pl.BlockSpec((BM, 128), lambda i: (i, 0))] * (nco + 2))
  return pl.pallas_call(
      body, grid=grid, in_specs=in_specs, out_specs=out_specs,
      out_shape=out_shape,
  )(*operands)


def _tc_mlp(pooled, w1, g1, b1, w2p, g2p, b2p, w3p, b3p):
  def body(p_ref, w1_ref, g1_ref, b1_ref, w2_ref, g2_ref, b2_ref, w3_ref,
           b3_ref, o_ref):
    h = jnp.dot(p_ref[...], w1_ref[...], preferred_element_type=_f32)
    h = _elu(h * INV_BN * g1_ref[...] + b1_ref[...])
    h = jnp.dot(h, w2_ref[...], preferred_element_type=_f32)
    h = _elu(h * INV_BN * g2_ref[...] + b2_ref[...])
    o_ref[...] = jnp.dot(h, w3_ref[...], preferred_element_type=_f32) + b3_ref[...]

  bs = lambda shp: pl.BlockSpec(shp, lambda: (0, 0))
  return pl.pallas_call(
      body,
      grid=(),
      in_specs=[bs((G, 512)), bs((512, 256)), bs((1, 256)), bs((1, 256)),
                bs((256, 128)), bs((1, 128)), bs((1, 128)), bs((128, 128)),
                bs((1, 128))],
      out_specs=bs((G, 128)),
      out_shape=jax.ShapeDtypeStruct((G, 128), _f32),
  )(pooled, w1, g1, b1, w2p, g2p, b2p, w3p, b3p)


def _make_sc_gat(nch, hh):
  mesh = plsc.VectorSubcoreMesh(core_axis_name="c", subcore_axis_name="s")

  def body(*refs):
    idx = 0
    hq = refs[idx:idx + nch]; idx += nch
    asp, adp, ssrc_r, sdst_r, meta_r = refs[idx:idx + 5]; idx += 5
    outs = refs[idx:idx + nch]; idx += nch
    (sidx, didx, agbuf, adl, s_slab, exbuf, rowbuf, mrow, sem) = \
        refs[idx:idx + 9]
    slabs = refs[idx + 9:]

    iota16 = lax.broadcasted_iota(_i32, (16,), 0)
    headmask = iota16 < hh
    zero16 = jnp.zeros((16,), _f32)
    w = lax.axis_index("s") * 2 + lax.axis_index("c")

    def strip(s_i, c0):
      st = w * 4 + s_i
      base = st * NS
      pltpu.sync_copy(meta_r.at[st], mrow)
      mv = mrow[...]
      ws = pl.multiple_of(mv[0], 8)
      cnt = mv[1]
      end = ws + cnt
      pltpu.sync_copy(adp.at[pl.ds(base, NS)], adl)

      def zs(i, c):
        s_slab[i] = zero16
        for fc in range(nch):
          for q in range(8):
            slabs[fc][i, pl.ds(q * 16, 16)] = zero16
        return c
      lax.fori_loop(0, NS, zs, 0)

      ntc = lax.div(cnt + KC - 1, KC)

      def tile(t, c):
        off = pl.multiple_of(ws + t * KC, 8)
        pltpu.sync_copy(ssrc_r.at[pl.ds(off, KC)], sidx)
        pltpu.sync_copy(sdst_r.at[pl.ds(off, KC)], didx)
        pltpu.async_copy(asp.at[sidx], agbuf, sem).wait()

        def grp_a(jb, c2):
          dvec = didx[pl.ds(jb * 16, 16)]
          lvec = dvec - base
          jg = off + jb * 16
          for k in range(16):
            j = jb * 16 + k
            l = lvec[k]
            valid = (l >= 0) & (l < NS) & (jg + k < end)
            lc = jnp.minimum(jnp.maximum(l, 0), NS - 1)
            v = agbuf[j, pl.ds(0, 16)] + adl[lc, pl.ds(0, 16)]
            e = jnp.maximum(v, 0.2 * v)
            ex = jnp.where(headmask, jnp.exp(e), zero16)
            ex = jnp.where(valid, ex, zero16)
            exbuf[j] = ex
            plsc.addupdate(s_slab.at[lc], ex)
          return c2
        lax.fori_loop(0, KC // 16, grp_a, 0)

        for fc in range(nch):
          pltpu.async_copy(hq[fc].at[sidx], rowbuf, sem).wait()

          def grp_b(jb, c2, fc=fc):
            dvec = didx[pl.ds(jb * 16, 16)]
            lvec = dvec - base
            for k in range(16):
              j = jb * 16 + k
              l = lvec[k]
              lc = jnp.minimum(jnp.maximum(l, 0), NS - 1)
              exv = exbuf[j]
              if hh == 8:
                w0 = exv[2 * fc]
                w1 = exv[2 * fc + 1]
              else:
                w0 = exv[0]
                w1 = w0
              for q in range(8):
                wq = w0 if q < 4 else w1
                plsc.addupdate(slabs[fc].at[lc, pl.ds(q * 16, 16)],
                               rowbuf[j, pl.ds(q * 16, 16)] * wq)
            return c2
          lax.fori_loop(0, KC // 16, grp_b, 0)
        return c
      lax.fori_loop(0, ntc, tile, 0)

      def norm(i, c):
        inv = 1.0 / (s_slab[i] + 1e-16)
        for fc in range(nch):
          for q in range(8):
            hqi = (2 * fc + (0 if q < 4 else 1)) if hh == 8 else 0
            slabs[fc][i, pl.ds(q * 16, 16)] = (
                slabs[fc][i, pl.ds(q * 16, 16)] * inv[hqi])
        return c
      lax.fori_loop(0, NS, norm, 0)
      for fc in range(nch):
        pltpu.sync_copy(slabs[fc], outs[fc].at[pl.ds(base, NS)])
      return c0

    lax.fori_loop(0, 4, strip, 0)

  out_type = [jax.ShapeDtypeStruct((NPAD, 128), _f32) for _ in range(nch)]
  scratch = [
      pltpu.VMEM((KC,), _i32),
      pltpu.VMEM((KC,), _i32),
      pltpu.VMEM((KC, 128), _f32),
      pltpu.VMEM((NS, 128), _f32),
      pltpu.VMEM((NS, 16), _f32),
      pltpu.VMEM((KC, 16), _f32),
      pltpu.VMEM((KC, 128), _f32),
      pltpu.VMEM((16,), _i32),
      pltpu.SemaphoreType.DMA,
  ] + [pltpu.VMEM((NS, 128), _f32) for _ in range(nch)]
  return pl.kernel(body, out_type=out_type, mesh=mesh, scratch_types=scratch)


def _sc_pool(agg0, agg1, b4v, g4v, bb4v, bptr):
  mesh = plsc.VectorSubcoreMesh(core_axis_name="c", subcore_axis_name="s")

  def body(a0_r, a1_r, b4_r, g4_r, bb4_r, bp_r, out_r,
           pvec, avec, dvec, buf0, buf1, poolbuf, sem):
    w = lax.axis_index("s") * 2 + lax.axis_index("c")
    pltpu.sync_copy(bp_r.at[pl.ds(w * 8, 16)], pvec)
    pltpu.sync_copy(b4_r, dvec)
    pltpu.sync_copy(g4_r, avec)
    pltpu.sync_copy(bb4_r, poolbuf.at[0, pl.ds(0, 256)])
    for q in range(16):
      a = avec[pl.ds(q * 16, 16)] * INV_BN
      d = a * dvec[pl.ds(q * 16, 16)] + poolbuf[0, pl.ds(q * 16, 16)]
      avec[pl.ds(q * 16, 16)] = a
      dvec[pl.ds(q * 16, 16)] = d

    bounds = pvec[...]
    neg = jnp.full((16,), -3.0e38, _f32)
    zero16 = jnp.zeros((16,), _f32)
    for g in range(8):
      b0 = bounds[g]
      b1 = bounds[g + 1]
      start = pl.multiple_of(lax.div(b0, 8) * 8, 8)
      nchunks = lax.div(b1 - start + KN - 1, KN)

      def chunk(t, carry, b0=b0, b1=b1, start=start):
        accs = carry
        off = pl.multiple_of(start + t * KN, 8)
        pltpu.sync_copy(a0_r.at[pl.ds(off, KN)], buf0)
        pltpu.sync_copy(a1_r.at[pl.ds(off, KN)], buf1)

        def node(j, carry2, off=off, b0=b0, b1=b1):
          accs2 = list(carry2)
          n = off + j
          valid = (n >= b0) & (n < b1)
          for q in range(16):
            src = buf0 if q < 8 else buf1
            qq = q if q < 8 else q - 8
            v = src[j, pl.ds(qq * 16, 16)]
            x = avec[pl.ds(q * 16, 16)] * v + dvec[pl.ds(q * 16, 16)]
            x = jnp.where(x > 0, x, jnp.exp(x) - 1.0)
            accs2[q] = jnp.where(valid, accs2[q] + x, accs2[q])
            accs2[16 + q] = jnp.where(valid, jnp.maximum(accs2[16 + q], x),
                                      accs2[16 + q])
          return tuple(accs2)
        return lax.fori_loop(0, KN, node, accs)

      init = tuple([zero16] * 16 + [neg] * 16)
      accs = lax.fori_loop(0, nchunks, chunk, init)
      cntf = (b1 - b0).astype(_f32)
      rinv = jnp.ones((16,), _f32) / jnp.maximum(cntf, 1.0)
      has = b1 > b0
      for q in range(16):
        poolbuf[g, pl.ds(q * 16, 16)] = accs[q] * rinv
        mx = jnp.where(has, accs[16 + q], zero16)
        poolbuf[g, pl.ds(256 + q * 16, 16)] = mx
    pltpu.sync_copy(poolbuf, out_r.at[pl.ds(w * 8, 8)])

  scratch = [
      pltpu.VMEM((16,), _i32),
      pltpu.VMEM((256,), _f32),
      pltpu.VMEM((256,), _f32),
      pltpu.VMEM((KN, 128), _f32),
      pltpu.VMEM((KN, 128), _f32),
      pltpu.VMEM((8, 512), _f32),
      pltpu.SemaphoreType.DMA,
  ]
  fn = pl.kernel(body, out_type=jax.ShapeDtypeStruct((G, 512), _f32),
                 mesh=mesh, scratch_types=scratch)
  return fn(agg0, agg1, b4v, g4v, bb4v, bptr)


def _sel_matrix(hc, heads):
  s = np.zeros((hc, 128), np.float32)
  ch = hc // heads
  for h in range(heads):
    s[h * ch:(h + 1) * ch, h] = 1.0
  return jnp.asarray(s)


def kernel(x, params, edge_index, batch):
  p = params
  src = edge_index[0].astype(_i32)
  dst = edge_index[1].astype(_i32)
  perm = jnp.argsort(dst)
  ssrc = src[perm]
  sdst = dst[perm]
  ssrc_p = jnp.pad(ssrc, (0, EPAD - E))
  sdst_p = jnp.pad(sdst, (0, EPAD - E))
  bounds = jnp.searchsorted(sdst, jnp.arange(NSTRIP + 1, dtype=_i32) * NS
                            ).astype(_i32)
  wstart = (bounds[:-1] // 8) * 8
  wend = ((bounds[1:] + 7) // 8) * 8
  meta = jnp.zeros((NSTRIP, 16), _i32)
  meta = meta.at[:, 0].set(wstart).at[:, 1].set(wend - wstart)
  bptr = jnp.searchsorted(batch.astype(_i32),
                          jnp.arange(G + 1, dtype=_i32)).astype(_i32)
  bptr = jnp.pad(bptr, (0, 15))

  xp = jnp.zeros((NPAD, 128), _f32).at[:N, :6].set(x)
  w0 = jnp.zeros((128, 128), _f32).at[:6, :64].set(p['Wp'])
  w1p = jnp.zeros((128, HC3), _f32).at[:64, :].set(p['W1'])
  g0 = jnp.zeros((1, 128), _f32).at[0, :64].set(p['bn0_g'])
  b0 = jnp.zeros((1, 128), _f32).at[0, :64].set(p['bn0_b'])
  sel3 = _sel_matrix(HC3, 8)
  sel4 = _sel_matrix(HC4, 1)

  def flat(a):
    return a.reshape(1, -1)

  res = _tc_first(xp, w0, w1p, flat(p['as1']), flat(p['ad1']), sel3, g0, b0)
  hq = list(res[:4])
  as_p, ad_p = res[4], res[5]

  sc3 = _make_sc_gat(4, 8)
  sc4 = _make_sc_gat(2, 1)

  xi = None
  for layer in (1, 2, 3):
    aggs = list(sc3(*hq, as_p, ad_p, ssrc_p, sdst_p, meta))
    nxt = layer + 1
    hcout = HC3 if nxt < 4 else HC4
    nco = 4 if nxt < 4 else 2
    sel = sel3 if nxt < 4 else sel4
    res = _tc_mid(aggs, xi, flat(p['b%d' % layer]),
                  flat(p['bn%d_g' % layer]), flat(p['bn%d_b' % layer]),
                  p['W%d' % nxt], flat(p['as%d' % nxt]),
                  flat(p['ad%d' % nxt]), sel, hcout, nco)
    xi = res[0]
    hq = list(res[1:1 + nco])
    as_p, ad_p = res[1 + nco], res[2 + nco]

  agg4 = list(sc4(*hq, as_p, ad_p, ssrc_p, sdst_p, meta))
  pooled = _sc_pool(agg4[0], agg4[1], p['b4'].reshape(-1),
                    p['bn4_g'].reshape(-1), p['bn4_b'].reshape(-1), bptr)

  w2p = jnp.zeros((256, 128), _f32).at[:, :64].set(p['fc2'])
  g2p = jnp.zeros((1, 128), _f32).at[0, :64].set(p['bnf2_g'])
  b2p = jnp.zeros((1, 128), _f32).at[0, :64].set(p['bnf2_b'])
  w3p = jnp.zeros((128, 128), _f32).at[:64, :2].set(p['fc3'])
  b3p = jnp.zeros((1, 128), _f32).at[0, :2].set(p['fc3_b'])
  out = _tc_mlp(pooled, p['fc1'], flat(p['bnf1_g']), flat(p['bnf1_b']),
                w2p, g2p, b2p, w3p, b3p)
  return out[:, :2]
